# baseline (speedup 1.0000x reference)
import functools

import jax
import jax.numpy as jnp
from jax import lax
from jax.experimental import pallas as pl
from jax.experimental.pallas import tpu as pltpu
from jax.experimental.pallas import tpu_sc as plsc

B = 16384
F = 32
NC = 2
NS = 16
NW = NC * NS
BPW = B // NW
L = 16
TS = 8
TL = 128
HALF = F // 2
NWAVE = BPW // L
PROWS = L * HALF


def _gmf_body(users_hbm, items_hbm, utab_hbm, itab_hbm, params_hbm, out_hbm,
              idx_u, idx_i, pan_u0, pan_u1, pan_u2, pan_i0, pan_i1, pan_i2,
              params_v, out_v, sem_u0, sem_u1, sem_u2, sem_i0, sem_i1,
              sem_i2):
    pan_u = (pan_u0, pan_u1, pan_u2)
    pan_i = (pan_i0, pan_i1, pan_i2)
    sem_u = (sem_u0, sem_u1, sem_u2)
    sem_i = (sem_i0, sem_i1, sem_i2)
    wid = lax.axis_index("s") * NC + lax.axis_index("c")
    base = wid * BPW

    pltpu.sync_copy(users_hbm.at[pl.ds(base, BPW)], idx_u)
    pltpu.sync_copy(items_hbm.at[pl.ds(base, BPW)], idx_i)
    pltpu.sync_copy(params_hbm, params_v)

    wv = [params_v[pl.ds(k * L, L)] for k in range(F // L)]
    bv = params_v[pl.ds(F // L * L, L)]
    w = [wv[f // L][f % L] for f in range(F)]
    bias = bv[0]
    lane = lax.iota(jnp.int32, L)

    def wave(v, carry):
        uvec = idx_u[pl.ds(v * L, L)]
        ivec = idx_i[pl.ds(v * L, L)]
        rem_u = uvec - (uvec // TL) * TL
        rem_i = ivec - (ivec // TL) * TL
        qus = [pl.multiple_of((uvec[k] // TL) * TL, TL) for k in range(L)]
        qis = [pl.multiple_of((ivec[k] // TL) * TL, TL) for k in range(L)]

        def issue(p):
            fr = p * TS
            pu, pi = pan_u[p % 3], pan_i[p % 3]
            su, si = sem_u[p % 3], sem_i[p % 3]
            cs = []
            for k in range(L):
                cs.append(pltpu.async_copy(
                    utab_hbm.at[pl.ds(fr, TS), pl.ds(qus[k], TL)],
                    pu.at[pl.ds(k * TS, TS)], su))
                cs.append(pltpu.async_copy(
                    itab_hbm.at[pl.ds(fr, TS), pl.ds(qis[k], TL)],
                    pi.at[pl.ds(k * TS, TS)], si))
            return cs

        acc = jnp.zeros((L,), jnp.float32)
        pending = [issue(0), issue(1)]
        for p in range(F // TS):
            if p + 2 < F // TS:
                pending.append(issue(p + 2))
            for c in pending.pop(0):
                c.wait()
            for fo in range(TS):
                f = p * TS + fo
                rows = lane * TS + fo
                ucol = plsc.load_gather(pan_u[p % 3], [rows, rem_u])
                icol = plsc.load_gather(pan_i[p % 3], [rows, rem_i])
                acc = acc + (ucol * icol) * w[f]
        z = acc + bias
        out_v[pl.ds(v * L, L)] = 1.0 / (1.0 + jnp.exp(-z))
        return carry

    lax.fori_loop(0, NWAVE, wave, 0)

    pltpu.sync_copy(out_v, out_hbm.at[pl.ds(base, BPW)])


_gmf = functools.partial(
    pl.kernel,
    out_type=jax.ShapeDtypeStruct((B,), jnp.float32),
    mesh=plsc.VectorSubcoreMesh(core_axis_name="c", subcore_axis_name="s"),
    scratch_types=[
        pltpu.VMEM((BPW,), jnp.int32),
        pltpu.VMEM((BPW,), jnp.int32),
        pltpu.VMEM((L * TS, TL), jnp.float32),
        pltpu.VMEM((L * TS, TL), jnp.float32),
        pltpu.VMEM((L * TS, TL), jnp.float32),
        pltpu.VMEM((L * TS, TL), jnp.float32),
        pltpu.VMEM((L * TS, TL), jnp.float32),
        pltpu.VMEM((L * TS, TL), jnp.float32),
        pltpu.VMEM((48,), jnp.float32),
        pltpu.VMEM((BPW,), jnp.float32),
        pltpu.SemaphoreType.DMA,
        pltpu.SemaphoreType.DMA,
        pltpu.SemaphoreType.DMA,
        pltpu.SemaphoreType.DMA,
        pltpu.SemaphoreType.DMA,
        pltpu.SemaphoreType.DMA,
    ],
    compiler_params=pltpu.CompilerParams(needs_layout_passes=False),
)(_gmf_body)


def kernel(users, items, user_table, item_table, pred_w, pred_b):
    params = jnp.concatenate([
        pred_w.reshape(-1).astype(jnp.float32),
        pred_b.reshape(-1).astype(jnp.float32),
        jnp.zeros((48 - F - 1,), jnp.float32),
    ])
    return _gmf(users.astype(jnp.int32), items.astype(jnp.int32),
                user_table.T, item_table.T, params)
